# Initial kernel scaffold; baseline (speedup 1.0000x reference)
import jax
import jax.numpy as jnp
from jax.experimental import pallas as pl


def kernel(x, W, rand_u):
    raise NotImplementedError("write your pallas kernel here")


# speedup vs baseline: 1.2102x; 1.2102x over previous
import functools

import jax
import jax.numpy as jnp
from jax import lax
from jax.experimental import pallas as pl
from jax.experimental.pallas import tpu as pltpu


def _reduce_body(x_ref, avg_ref, max_ref, *, inv_c):
    xb = x_ref[0]
    avg_ref[0, 0] = jnp.sum(xb, axis=0) * inv_c
    max_ref[0, 0] = jnp.max(xb, axis=0)


def _channel_reduce(xr, L):
    B, C, HW = xr.shape
    nb = HW // L
    out_sd = jax.ShapeDtypeStruct((B, 1, HW), jnp.float32)
    return pl.pallas_call(
        functools.partial(_reduce_body, inv_c=1.0 / C),
        grid=(B, nb),
        in_specs=[pl.BlockSpec((1, C, L), lambda b, i: (b, 0, i))],
        out_specs=[
            pl.BlockSpec((1, 1, L), lambda b, i: (b, 0, i)),
            pl.BlockSpec((1, 1, L), lambda b, i: (b, 0, i)),
        ],
        out_shape=[out_sd, out_sd],
    )(xr)


def _bf16_rne(v):
    u = lax.bitcast_convert_type(v, jnp.int32)
    r = (u + jnp.int32(0x7FFF) + ((u >> 16) & jnp.int32(1))) & jnp.int32(-65536)
    return lax.bitcast_convert_type(r, jnp.float32)


def _map_body(w_ref, avg_ref, max_ref, rand_ref, fm_ref, *, H, W, kth):
    a = _bf16_rne(avg_ref[0])
    m = _bf16_rne(max_ref[0])
    pa = jnp.pad(a, ((1, 1), (1, 1)))
    pm = jnp.pad(m, ((1, 1), (1, 1)))
    c = jnp.zeros((H, W), jnp.float32)
    for dy in range(3):
        for dx in range(3):
            c = c + _bf16_rne(w_ref[0, dy * 3 + dx]) * pa[dy:dy + H, dx:dx + W]
            c = c + _bf16_rne(w_ref[0, 9 + dy * 3 + dx]) * pm[dy:dy + H, dx:dx + W]

    s = lax.bitcast_convert_type(c, jnp.int32)
    key = jnp.where(s >= 0, s, s ^ jnp.int32(0x7FFFFFFF))
    sign = jnp.int32(-2147483648)

    def body(i, biased):
        bit = jnp.left_shift(jnp.int32(1), 31 - i)
        cand_b = biased | bit
        cand = cand_b ^ sign
        cnt = jnp.sum((key >= cand).astype(jnp.int32))
        return jnp.where(cnt >= kth, cand_b, biased)

    biased = lax.fori_loop(0, 32, body, jnp.int32(0))
    kkey = biased ^ sign
    s_th = jnp.where(kkey >= 0, kkey, kkey ^ jnp.int32(0x7FFFFFFF))
    thresh = lax.bitcast_convert_type(s_th, jnp.float32)

    x_iptc = jax.nn.sigmoid(c)
    x_rand = jnp.where(rand_ref[0] - 0.5 < 0, 1.0, 0.0)
    maskv = jnp.where(c - thresh < 0, 1.0, 0.0)
    fm_ref[0] = maskv * x_rand * x_iptc + x_iptc * (1.0 - x_rand)


def _make_map(avg, mx, wf, rand_u, kth):
    B, _, HW = avg.shape
    H = W = int(round(HW ** 0.5))
    a3 = avg.reshape(B, H, W)
    m3 = mx.reshape(B, H, W)
    return pl.pallas_call(
        functools.partial(_map_body, H=H, W=W, kth=kth),
        grid=(B,),
        in_specs=[
            pl.BlockSpec(memory_space=pltpu.SMEM),
            pl.BlockSpec((1, H, W), lambda b: (b, 0, 0)),
            pl.BlockSpec((1, H, W), lambda b: (b, 0, 0)),
            pl.BlockSpec((1, H, W), lambda b: (b, 0, 0)),
        ],
        out_specs=pl.BlockSpec((1, H, W), lambda b: (b, 0, 0)),
        out_shape=jax.ShapeDtypeStruct((B, H, W), jnp.float32),
    )(wf, a3, m3, rand_u)


def _mul_body(x_ref, fm_ref, o_ref):
    o_ref[0] = x_ref[0] * fm_ref[0, 0][None, :]


def _apply_map(xr, fm3, L):
    B, C, HW = xr.shape
    nb = HW // L
    return pl.pallas_call(
        _mul_body,
        grid=(B, nb),
        in_specs=[
            pl.BlockSpec((1, C, L), lambda b, i: (b, 0, i)),
            pl.BlockSpec((1, 1, L), lambda b, i: (b, 0, i)),
        ],
        out_specs=pl.BlockSpec((1, C, L), lambda b, i: (b, 0, i)),
        out_shape=jax.ShapeDtypeStruct((B, C, HW), jnp.float32),
    )(xr, fm3)


def kernel(x, W, rand_u):
    B, C, H, Wd = x.shape
    HW = H * Wd
    p = 0.8
    kth = int((1.0 - p) * HW)

    xr = x.reshape(B, C, HW)
    L = 6272 if HW % 6272 == 0 else HW

    avg, mx = _channel_reduce(xr, L)
    wf = W.reshape(1, 18)
    fm = _make_map(avg, mx, wf, rand_u, kth)
    fm3 = fm.reshape(B, 1, HW)
    out = _apply_map(xr, fm3, L)
    return out.reshape(B, C, H, Wd)


# speedup vs baseline: 1.2199x; 1.0081x over previous
import functools

import jax
import jax.numpy as jnp
from jax import lax
from jax.experimental import pallas as pl
from jax.experimental.pallas import tpu as pltpu


def _reduce_body(x_ref, avg_ref, max_ref, *, inv_c):
    xb = x_ref[0]
    avg_ref[0, 0] = jnp.sum(xb, axis=0) * inv_c
    max_ref[0, 0] = jnp.max(xb, axis=0)


def _reduce_contig_body(x_ref, avg_ref, max_ref, *, inv_c, nc):
    c = pl.program_id(1)
    xb = x_ref[0]
    s = jnp.sum(xb, axis=0)
    m = jnp.max(xb, axis=0)

    @pl.when(c == 0)
    def _init():
        avg_ref[0, 0] = s
        max_ref[0, 0] = m

    @pl.when(c > 0)
    def _acc():
        avg_ref[0, 0] += s
        max_ref[0, 0] = jnp.maximum(max_ref[0, 0], m)

    @pl.when(c == nc - 1)
    def _fin():
        avg_ref[0, 0] = avg_ref[0, 0] * inv_c


def _channel_reduce_contig(xr, CC):
    B, C, HW = xr.shape
    nc = C // CC
    out_sd = jax.ShapeDtypeStruct((B, 1, HW), jnp.float32)
    return pl.pallas_call(
        functools.partial(_reduce_contig_body, inv_c=1.0 / C, nc=nc),
        grid=(B, nc),
        in_specs=[pl.BlockSpec((1, CC, HW), lambda b, c: (b, c, 0))],
        out_specs=[
            pl.BlockSpec((1, 1, HW), lambda b, c: (b, 0, 0)),
            pl.BlockSpec((1, 1, HW), lambda b, c: (b, 0, 0)),
        ],
        out_shape=[out_sd, out_sd],
        compiler_params=pltpu.CompilerParams(
            dimension_semantics=("arbitrary", "arbitrary")),
    )(xr)


def _reduce2_body(x1_ref, x2_ref, avg_ref, max_ref, *, inv_c):
    a = x1_ref[0, 0]
    b = x2_ref[0, 0]
    avg_ref[0, 0] = (jnp.sum(a, axis=0) + jnp.sum(b, axis=0)) * inv_c
    max_ref[0, 0] = jnp.maximum(jnp.max(a, axis=0), jnp.max(b, axis=0))


def _channel_reduce2(x4, L):
    B, S, C2, HW = x4.shape
    nb = HW // L
    out_sd = jax.ShapeDtypeStruct((B, 1, HW), jnp.float32)
    return pl.pallas_call(
        functools.partial(_reduce2_body, inv_c=1.0 / (S * C2)),
        grid=(B, nb),
        in_specs=[
            pl.BlockSpec((1, 1, C2, L), lambda b, i: (b, 0, 0, i)),
            pl.BlockSpec((1, 1, C2, L), lambda b, i: (b, 1, 0, i)),
        ],
        out_specs=[
            pl.BlockSpec((1, 1, L), lambda b, i: (b, 0, i)),
            pl.BlockSpec((1, 1, L), lambda b, i: (b, 0, i)),
        ],
        out_shape=[out_sd, out_sd],
        compiler_params=pltpu.CompilerParams(
            dimension_semantics=("parallel", "parallel")),
    )(x4, x4)


def _channel_reduce(xr, L):
    B, C, HW = xr.shape
    nb = HW // L
    out_sd = jax.ShapeDtypeStruct((B, 1, HW), jnp.float32)
    return pl.pallas_call(
        functools.partial(_reduce_body, inv_c=1.0 / C),
        grid=(B, nb),
        in_specs=[pl.BlockSpec((1, C, L), lambda b, i: (b, 0, i))],
        out_specs=[
            pl.BlockSpec((1, 1, L), lambda b, i: (b, 0, i)),
            pl.BlockSpec((1, 1, L), lambda b, i: (b, 0, i)),
        ],
        out_shape=[out_sd, out_sd],
        compiler_params=pltpu.CompilerParams(
            dimension_semantics=("parallel", "parallel")),
    )(xr)


def _fused_body(w_ref, rand_ref, x_ref, fm_ref, sacc_ref, macc_ref,
                *, nc, inv_c, H, W, kth):
    c = pl.program_id(1)
    xb = x_ref[0]
    s = jnp.sum(xb, axis=0)
    m = jnp.max(xb, axis=0)

    @pl.when(c == 0)
    def _init():
        sacc_ref[...] = s
        macc_ref[...] = m

    @pl.when(c > 0)
    def _acc():
        sacc_ref[...] += s
        macc_ref[...] = jnp.maximum(macc_ref[...], m)

    @pl.when(c == nc - 1)
    def _map():
        a = _bf16_rne(sacc_ref[...] * inv_c)
        mm = _bf16_rne(macc_ref[...])
        pa = jnp.pad(a, ((1, 1), (1, 1)))
        pm = jnp.pad(mm, ((1, 1), (1, 1)))
        cv = jnp.zeros((H, W), jnp.float32)
        for dy in range(3):
            for dx in range(3):
                cv = cv + _bf16_rne(w_ref[0, dy * 3 + dx]) * pa[dy:dy + H, dx:dx + W]
                cv = cv + _bf16_rne(w_ref[0, 9 + dy * 3 + dx]) * pm[dy:dy + H, dx:dx + W]
        sb = lax.bitcast_convert_type(cv, jnp.int32)
        key = jnp.where(sb >= 0, sb, sb ^ jnp.int32(0x7FFFFFFF))
        sign = jnp.int32(-2147483648)

        def body(i, biased):
            bit = jnp.left_shift(jnp.int32(1), 31 - i)
            cand_b = biased | bit
            cand = cand_b ^ sign
            cnt = jnp.sum((key >= cand).astype(jnp.int32))
            return jnp.where(cnt >= kth, cand_b, biased)

        biased = lax.fori_loop(0, 32, body, jnp.int32(0))
        kkey = biased ^ sign
        s_th = jnp.where(kkey >= 0, kkey, kkey ^ jnp.int32(0x7FFFFFFF))
        thresh = lax.bitcast_convert_type(s_th, jnp.float32)

        x_iptc = jax.nn.sigmoid(cv)
        x_rand = jnp.where(rand_ref[0] - 0.5 < 0, 1.0, 0.0)
        maskv = jnp.where(cv - thresh < 0, 1.0, 0.0)
        fm_ref[0] = maskv * x_rand * x_iptc + x_iptc * (1.0 - x_rand)


def _fused_reduce_map(x, wf, rand_u, kth, CC):
    B, C, H, W = x.shape
    nc = C // CC
    return pl.pallas_call(
        functools.partial(_fused_body, nc=nc, inv_c=1.0 / C, H=H, W=W, kth=kth),
        grid=(B, nc),
        in_specs=[
            pl.BlockSpec(memory_space=pltpu.SMEM),
            pl.BlockSpec((1, H, W), lambda b, c: (b, 0, 0)),
            pl.BlockSpec((1, CC, H, W), lambda b, c: (b, c, 0, 0)),
        ],
        out_specs=pl.BlockSpec((1, H, W), lambda b, c: (b, 0, 0)),
        out_shape=jax.ShapeDtypeStruct((B, H, W), jnp.float32),
        scratch_shapes=[
            pltpu.VMEM((H, W), jnp.float32),
            pltpu.VMEM((H, W), jnp.float32),
        ],
        compiler_params=pltpu.CompilerParams(
            dimension_semantics=("arbitrary", "arbitrary")),
    )(wf, rand_u, x)


def _bf16_rne(v):
    u = lax.bitcast_convert_type(v, jnp.int32)
    r = (u + jnp.int32(0x7FFF) + ((u >> 16) & jnp.int32(1))) & jnp.int32(-65536)
    return lax.bitcast_convert_type(r, jnp.float32)


def _map_body(w_ref, avg_ref, max_ref, rand_ref, fm_ref, *, H, W, kth):
    a = _bf16_rne(avg_ref[0])
    m = _bf16_rne(max_ref[0])
    pa = jnp.pad(a, ((1, 1), (1, 1)))
    pm = jnp.pad(m, ((1, 1), (1, 1)))
    c = jnp.zeros((H, W), jnp.float32)
    for dy in range(3):
        for dx in range(3):
            c = c + _bf16_rne(w_ref[0, dy * 3 + dx]) * pa[dy:dy + H, dx:dx + W]
            c = c + _bf16_rne(w_ref[0, 9 + dy * 3 + dx]) * pm[dy:dy + H, dx:dx + W]

    s = lax.bitcast_convert_type(c, jnp.int32)
    key = jnp.where(s >= 0, s, s ^ jnp.int32(0x7FFFFFFF))
    sign = jnp.int32(-2147483648)

    def body(i, biased):
        bit = jnp.left_shift(jnp.int32(1), 31 - i)
        cand_b = biased | bit
        cand = cand_b ^ sign
        cnt = jnp.sum((key >= cand).astype(jnp.int32))
        return jnp.where(cnt >= kth, cand_b, biased)

    biased = lax.fori_loop(0, 32, body, jnp.int32(0))
    kkey = biased ^ sign
    s_th = jnp.where(kkey >= 0, kkey, kkey ^ jnp.int32(0x7FFFFFFF))
    thresh = lax.bitcast_convert_type(s_th, jnp.float32)

    x_iptc = jax.nn.sigmoid(c)
    x_rand = jnp.where(rand_ref[0] - 0.5 < 0, 1.0, 0.0)
    maskv = jnp.where(c - thresh < 0, 1.0, 0.0)
    fm_ref[0] = maskv * x_rand * x_iptc + x_iptc * (1.0 - x_rand)


def _make_map(avg, mx, wf, rand_u, kth):
    B, _, HW = avg.shape
    H = W = int(round(HW ** 0.5))
    a3 = avg.reshape(B, H, W)
    m3 = mx.reshape(B, H, W)
    return pl.pallas_call(
        functools.partial(_map_body, H=H, W=W, kth=kth),
        grid=(B,),
        in_specs=[
            pl.BlockSpec(memory_space=pltpu.SMEM),
            pl.BlockSpec((1, H, W), lambda b: (b, 0, 0)),
            pl.BlockSpec((1, H, W), lambda b: (b, 0, 0)),
            pl.BlockSpec((1, H, W), lambda b: (b, 0, 0)),
        ],
        out_specs=pl.BlockSpec((1, H, W), lambda b: (b, 0, 0)),
        out_shape=jax.ShapeDtypeStruct((B, H, W), jnp.float32),
    )(wf, a3, m3, rand_u)


def _mul_body(x_ref, fm_ref, o_ref):
    o_ref[0] = x_ref[0] * fm_ref[0, 0][None, :]


def _apply_map(xr, fm3, L):
    B, C, HW = xr.shape
    nb = HW // L
    return pl.pallas_call(
        _mul_body,
        grid=(B, nb),
        in_specs=[
            pl.BlockSpec((1, C, L), lambda b, i: (b, 0, i)),
            pl.BlockSpec((1, 1, L), lambda b, i: (b, 0, i)),
        ],
        out_specs=pl.BlockSpec((1, C, L), lambda b, i: (b, 0, i)),
        out_shape=jax.ShapeDtypeStruct((B, C, HW), jnp.float32),
    )(xr, fm3)


def kernel(x, W, rand_u):
    B, C, H, Wd = x.shape
    HW = H * Wd
    p = 0.8
    kth = int((1.0 - p) * HW)

    xr = x.reshape(B, C, HW)
    L = 6272 if HW % 6272 == 0 else HW

    wf = W.reshape(1, 18)
    fm = _fused_reduce_map(x, wf, rand_u, kth, 24 if C % 24 == 0 else C)
    fm3 = fm.reshape(B, 1, HW)
    out = _apply_map(xr, fm3, 12544)
    return out.reshape(B, C, H, Wd)


# speedup vs baseline: 3.2567x; 2.6696x over previous
import functools

import jax
import jax.numpy as jnp
from jax import lax
from jax.experimental import pallas as pl
from jax.experimental.pallas import tpu as pltpu


def _reduce_body(x_ref, avg_ref, max_ref, *, inv_c):
    xb = x_ref[0]
    avg_ref[0, 0] = jnp.sum(xb, axis=0) * inv_c
    max_ref[0, 0] = jnp.max(xb, axis=0)


def _reduce_contig_body(x_ref, avg_ref, max_ref, *, inv_c, nc):
    c = pl.program_id(1)
    xb = x_ref[0]
    s = jnp.sum(xb, axis=0)
    m = jnp.max(xb, axis=0)

    @pl.when(c == 0)
    def _init():
        avg_ref[0, 0] = s
        max_ref[0, 0] = m

    @pl.when(c > 0)
    def _acc():
        avg_ref[0, 0] += s
        max_ref[0, 0] = jnp.maximum(max_ref[0, 0], m)

    @pl.when(c == nc - 1)
    def _fin():
        avg_ref[0, 0] = avg_ref[0, 0] * inv_c


def _channel_reduce_contig(xr, CC):
    B, C, HW = xr.shape
    nc = C // CC
    out_sd = jax.ShapeDtypeStruct((B, 1, HW), jnp.float32)
    return pl.pallas_call(
        functools.partial(_reduce_contig_body, inv_c=1.0 / C, nc=nc),
        grid=(B, nc),
        in_specs=[pl.BlockSpec((1, CC, HW), lambda b, c: (b, c, 0))],
        out_specs=[
            pl.BlockSpec((1, 1, HW), lambda b, c: (b, 0, 0)),
            pl.BlockSpec((1, 1, HW), lambda b, c: (b, 0, 0)),
        ],
        out_shape=[out_sd, out_sd],
        compiler_params=pltpu.CompilerParams(
            dimension_semantics=("arbitrary", "arbitrary")),
    )(xr)


def _reduce2_body(x1_ref, x2_ref, avg_ref, max_ref, *, inv_c):
    a = x1_ref[0, 0]
    b = x2_ref[0, 0]
    avg_ref[0, 0] = (jnp.sum(a, axis=0) + jnp.sum(b, axis=0)) * inv_c
    max_ref[0, 0] = jnp.maximum(jnp.max(a, axis=0), jnp.max(b, axis=0))


def _channel_reduce2(x4, L):
    B, S, C2, HW = x4.shape
    nb = HW // L
    out_sd = jax.ShapeDtypeStruct((B, 1, HW), jnp.float32)
    return pl.pallas_call(
        functools.partial(_reduce2_body, inv_c=1.0 / (S * C2)),
        grid=(B, nb),
        in_specs=[
            pl.BlockSpec((1, 1, C2, L), lambda b, i: (b, 0, 0, i)),
            pl.BlockSpec((1, 1, C2, L), lambda b, i: (b, 1, 0, i)),
        ],
        out_specs=[
            pl.BlockSpec((1, 1, L), lambda b, i: (b, 0, i)),
            pl.BlockSpec((1, 1, L), lambda b, i: (b, 0, i)),
        ],
        out_shape=[out_sd, out_sd],
        compiler_params=pltpu.CompilerParams(
            dimension_semantics=("parallel", "parallel")),
    )(x4, x4)


def _channel_reduce(xr, L):
    B, C, HW = xr.shape
    nb = HW // L
    out_sd = jax.ShapeDtypeStruct((B, 1, HW), jnp.float32)
    return pl.pallas_call(
        functools.partial(_reduce_body, inv_c=1.0 / C),
        grid=(B, nb),
        in_specs=[pl.BlockSpec((1, C, L), lambda b, i: (b, 0, i))],
        out_specs=[
            pl.BlockSpec((1, 1, L), lambda b, i: (b, 0, i)),
            pl.BlockSpec((1, 1, L), lambda b, i: (b, 0, i)),
        ],
        out_shape=[out_sd, out_sd],
        compiler_params=pltpu.CompilerParams(
            dimension_semantics=("parallel", "parallel")),
    )(xr)


def _fused_body(w_ref, rand_ref, x_ref, fm_ref, sacc_ref, macc_ref,
                *, nc, inv_c, H, W, kth):
    c = pl.program_id(1)
    xb = x_ref[0]
    s = jnp.sum(xb, axis=0)
    m = jnp.max(xb, axis=0)

    @pl.when(c == 0)
    def _init():
        sacc_ref[...] = s
        macc_ref[...] = m

    @pl.when(c > 0)
    def _acc():
        sacc_ref[...] += s
        macc_ref[...] = jnp.maximum(macc_ref[...], m)

    @pl.when(c == nc - 1)
    def _map():
        a = _bf16_rne(sacc_ref[...] * inv_c)
        mm = _bf16_rne(macc_ref[...])
        pa = jnp.pad(a, ((1, 1), (1, 1)))
        pm = jnp.pad(mm, ((1, 1), (1, 1)))
        cv = jnp.zeros((H, W), jnp.float32)
        for dy in range(3):
            for dx in range(3):
                cv = cv + _bf16_rne(w_ref[0, dy * 3 + dx]) * pa[dy:dy + H, dx:dx + W]
                cv = cv + _bf16_rne(w_ref[0, 9 + dy * 3 + dx]) * pm[dy:dy + H, dx:dx + W]
        sb = lax.bitcast_convert_type(cv, jnp.int32)
        key = jnp.where(sb >= 0, sb, sb ^ jnp.int32(0x7FFFFFFF))
        sign = jnp.int32(-2147483648)

        def body(i, biased):
            bit = jnp.left_shift(jnp.int32(1), 31 - i)
            cand_b = biased | bit
            cand = cand_b ^ sign
            cnt = jnp.sum((key >= cand).astype(jnp.int32))
            return jnp.where(cnt >= kth, cand_b, biased)

        biased = lax.fori_loop(0, 32, body, jnp.int32(0))
        kkey = biased ^ sign
        s_th = jnp.where(kkey >= 0, kkey, kkey ^ jnp.int32(0x7FFFFFFF))
        thresh = lax.bitcast_convert_type(s_th, jnp.float32)

        x_iptc = jax.nn.sigmoid(cv)
        x_rand = jnp.where(rand_ref[0] - 0.5 < 0, 1.0, 0.0)
        maskv = jnp.where(cv - thresh < 0, 1.0, 0.0)
        fm_ref[0] = maskv * x_rand * x_iptc + x_iptc * (1.0 - x_rand)


def _fused_reduce_map(x, wf, rand_u, kth, CC):
    B, C, H, W = x.shape
    nc = C // CC
    return pl.pallas_call(
        functools.partial(_fused_body, nc=nc, inv_c=1.0 / C, H=H, W=W, kth=kth),
        grid=(B, nc),
        in_specs=[
            pl.BlockSpec(memory_space=pltpu.SMEM),
            pl.BlockSpec((1, H, W), lambda b, c: (b, 0, 0)),
            pl.BlockSpec((1, CC, H, W), lambda b, c: (b, c, 0, 0)),
        ],
        out_specs=pl.BlockSpec((1, H, W), lambda b, c: (b, 0, 0)),
        out_shape=jax.ShapeDtypeStruct((B, H, W), jnp.float32),
        scratch_shapes=[
            pltpu.VMEM((H, W), jnp.float32),
            pltpu.VMEM((H, W), jnp.float32),
        ],
        compiler_params=pltpu.CompilerParams(
            dimension_semantics=("arbitrary", "arbitrary")),
    )(wf, rand_u, x)


def _bf16_rne(v):
    u = lax.bitcast_convert_type(v, jnp.int32)
    r = (u + jnp.int32(0x7FFF) + ((u >> 16) & jnp.int32(1))) & jnp.int32(-65536)
    return lax.bitcast_convert_type(r, jnp.float32)


def _map_body(w_ref, avg_ref, max_ref, rand_ref, fm_ref, *, H, W, kth):
    a = _bf16_rne(avg_ref[0])
    m = _bf16_rne(max_ref[0])
    pa = jnp.pad(a, ((1, 1), (1, 1)))
    pm = jnp.pad(m, ((1, 1), (1, 1)))
    c = jnp.zeros((H, W), jnp.float32)
    for dy in range(3):
        for dx in range(3):
            c = c + _bf16_rne(w_ref[0, dy * 3 + dx]) * pa[dy:dy + H, dx:dx + W]
            c = c + _bf16_rne(w_ref[0, 9 + dy * 3 + dx]) * pm[dy:dy + H, dx:dx + W]

    s = lax.bitcast_convert_type(c, jnp.int32)
    key = jnp.where(s >= 0, s, s ^ jnp.int32(0x7FFFFFFF))
    sign = jnp.int32(-2147483648)

    def body(i, biased):
        bit = jnp.left_shift(jnp.int32(1), 31 - i)
        cand_b = biased | bit
        cand = cand_b ^ sign
        cnt = jnp.sum((key >= cand).astype(jnp.int32))
        return jnp.where(cnt >= kth, cand_b, biased)

    biased = lax.fori_loop(0, 32, body, jnp.int32(0))
    kkey = biased ^ sign
    s_th = jnp.where(kkey >= 0, kkey, kkey ^ jnp.int32(0x7FFFFFFF))
    thresh = lax.bitcast_convert_type(s_th, jnp.float32)

    x_iptc = jax.nn.sigmoid(c)
    x_rand = jnp.where(rand_ref[0] - 0.5 < 0, 1.0, 0.0)
    maskv = jnp.where(c - thresh < 0, 1.0, 0.0)
    fm_ref[0] = maskv * x_rand * x_iptc + x_iptc * (1.0 - x_rand)


def _make_map(avg, mx, wf, rand_u, kth):
    B, _, HW = avg.shape
    H = W = int(round(HW ** 0.5))
    a3 = avg.reshape(B, H, W)
    m3 = mx.reshape(B, H, W)
    return pl.pallas_call(
        functools.partial(_map_body, H=H, W=W, kth=kth),
        grid=(B,),
        in_specs=[
            pl.BlockSpec(memory_space=pltpu.SMEM),
            pl.BlockSpec((1, H, W), lambda b: (b, 0, 0)),
            pl.BlockSpec((1, H, W), lambda b: (b, 0, 0)),
            pl.BlockSpec((1, H, W), lambda b: (b, 0, 0)),
        ],
        out_specs=pl.BlockSpec((1, H, W), lambda b: (b, 0, 0)),
        out_shape=jax.ShapeDtypeStruct((B, H, W), jnp.float32),
    )(wf, a3, m3, rand_u)


def _mul_body(x_ref, fm_ref, o_ref):
    o_ref[0] = x_ref[0] * fm_ref[0, 0][None, :]


def _mul4_body(x_ref, fm_ref, o_ref):
    o_ref[0] = x_ref[0] * fm_ref[0][None]


def _apply_map4(x, fm, CC):
    B, C, H, W = x.shape
    nc = C // CC
    return pl.pallas_call(
        _mul4_body,
        grid=(B, nc),
        in_specs=[
            pl.BlockSpec((1, CC, H, W), lambda b, c: (b, c, 0, 0)),
            pl.BlockSpec((1, H, W), lambda b, c: (b, 0, 0)),
        ],
        out_specs=pl.BlockSpec((1, CC, H, W), lambda b, c: (b, c, 0, 0)),
        out_shape=jax.ShapeDtypeStruct((B, C, H, W), jnp.float32),
        compiler_params=pltpu.CompilerParams(
            dimension_semantics=("parallel", "parallel")),
    )(x, fm)


def _apply_map(xr, fm3, L):
    B, C, HW = xr.shape
    nb = HW // L
    return pl.pallas_call(
        _mul_body,
        grid=(B, nb),
        in_specs=[
            pl.BlockSpec((1, C, L), lambda b, i: (b, 0, i)),
            pl.BlockSpec((1, 1, L), lambda b, i: (b, 0, i)),
        ],
        out_specs=pl.BlockSpec((1, C, L), lambda b, i: (b, 0, i)),
        out_shape=jax.ShapeDtypeStruct((B, C, HW), jnp.float32),
    )(xr, fm3)


def kernel(x, W, rand_u):
    B, C, H, Wd = x.shape
    HW = H * Wd
    p = 0.8
    kth = int((1.0 - p) * HW)

    xr = x.reshape(B, C, HW)
    L = 6272 if HW % 6272 == 0 else HW

    wf = W.reshape(1, 18)
    fm = _fused_reduce_map(x, wf, rand_u, kth, 24 if C % 24 == 0 else C)
    return _apply_map4(x, fm, 24 if C % 24 == 0 else C)
